# baseline (speedup 1.0000x reference)
import functools

import jax
import jax.numpy as jnp
from jax import lax
from jax.experimental import pallas as pl
from jax.experimental.pallas import tpu as pltpu
from jax.experimental.pallas import tpu_sc as plsc

N = 10000
D = 128
HF = 32
B = 64
C = 16

NW = 32
CH = N // NW
TAIL = N - NW * CH
TAIL_BASE = NW * CH
BR = 400
L = 16


def _xw_body(x_ref, w1_ref, xw_ref):
    xw_ref[...] = jnp.dot(x_ref[...], w1_ref[...],
                          preferred_element_type=jnp.float32,
                          precision=lax.Precision.HIGHEST)


def _main_body(f_ref, xw_ref, b1_ref, h_ref):
    acc = jnp.dot(f_ref[...], xw_ref[...],
                  preferred_element_type=jnp.float32,
                  precision=lax.Precision.HIGHEST)
    h_ref[...] = jnp.maximum(acc + b1_ref[...], 0.0)


def _final_body(p_ref, w2_ref, b2_ref, o_ref):
    pooled = jnp.max(p_ref[...], axis=0)
    o_ref[...] = jnp.dot(pooled, w2_ref[...],
                         preferred_element_type=jnp.float32,
                         precision=lax.Precision.HIGHEST) + b2_ref[...]


def _sc_pool_body(h_hbm, ind_hbm, part_hbm, rows_v, ind_v, rows_t, ind_t, pbuf, sem):
    wid = lax.axis_index("s") * 2 + lax.axis_index("c")
    base = wid * CH
    cp_rows = pltpu.async_copy(h_hbm.at[pl.ds(base, CH)], rows_v, sem)
    cp_ind = pltpu.async_copy(ind_hbm.at[pl.ds(base, CH)], ind_v, sem)

    zero = jnp.zeros((L,), jnp.float32)

    @pl.loop(0, B)
    def _(g):
        pbuf[g, pl.ds(0, L)] = zero
        pbuf[g, pl.ds(L, L)] = zero

    cp_rows.wait()
    cp_ind.wait()

    def rmw_row(g, rows, r):
        pbuf[g, pl.ds(0, L)] = jnp.maximum(pbuf[g, pl.ds(0, L)],
                                           rows[r, pl.ds(0, L)])
        pbuf[g, pl.ds(L, L)] = jnp.maximum(pbuf[g, pl.ds(L, L)],
                                           rows[r, pl.ds(L, L)])

    @pl.loop(0, (CH + L - 1) // L)
    def _(k):
        o = jnp.minimum(k * L, CH - L)
        gv = ind_v[pl.ds(o, L)]
        for j in range(L):
            rmw_row(gv[j], rows_v, o + j)

    @pl.when(wid == 0)
    def _():
        pltpu.async_copy(h_hbm.at[pl.ds(TAIL_BASE, TAIL)], rows_t, sem).wait()
        pltpu.async_copy(ind_hbm.at[pl.ds(TAIL_BASE, TAIL)], ind_t, sem).wait()
        gv = ind_t[pl.ds(0, L)]
        for j in range(TAIL):
            rmw_row(gv[j], rows_t, j)

    pltpu.async_copy(pbuf, part_hbm.at[wid], sem).wait()


@functools.cache
def _make_sc_pool():
    return pl.kernel(
        _sc_pool_body,
        out_type=jax.ShapeDtypeStruct((NW, B, HF), jnp.float32),
        mesh=plsc.VectorSubcoreMesh(core_axis_name="c", subcore_axis_name="s"),
        scratch_types=[
            pltpu.VMEM((CH, HF), jnp.float32),
            pltpu.VMEM((CH,), jnp.int32),
            pltpu.VMEM((TAIL, HF), jnp.float32),
            pltpu.VMEM((TAIL,), jnp.int32),
            pltpu.VMEM((B, HF), jnp.float32),
            pltpu.SemaphoreType.DMA,
        ],
    )


def kernel(filtre, X, node_indicator, W1, b1, W2, b2):
    xw = pl.pallas_call(
        _xw_body,
        out_shape=jax.ShapeDtypeStruct((N, HF), jnp.float32),
    )(X, W1)

    h = pl.pallas_call(
        _main_body,
        grid=(N // BR,),
        in_specs=[
            pl.BlockSpec((BR, N), lambda i: (i, 0)),
            pl.BlockSpec((N, HF), lambda i: (0, 0)),
            pl.BlockSpec((1, HF), lambda i: (0, 0)),
        ],
        out_specs=pl.BlockSpec((BR, HF), lambda i: (i, 0)),
        out_shape=jax.ShapeDtypeStruct((N, HF), jnp.float32),
    )(filtre, xw, b1.reshape(1, HF))

    part = _make_sc_pool()(h, node_indicator)

    out = pl.pallas_call(
        _final_body,
        out_shape=jax.ShapeDtypeStruct((B, C), jnp.float32),
    )(part, W2, b2.reshape(1, C))
    return out


# speedup vs baseline: 2.3482x; 2.3482x over previous
import functools

import jax
import jax.numpy as jnp
from jax import lax
from jax.experimental import pallas as pl
from jax.experimental.pallas import tpu as pltpu
from jax.experimental.pallas import tpu_sc as plsc

N = 10000
D = 128
HF = 32
B = 64
C = 16

NW = 32
CH = N // NW
TAIL = N - NW * CH
TAIL_BASE = NW * CH
BR = 400
L = 16


def _main_body(x_ref, w1_ref, f_ref, b1_ref, h_ref, xw_s):
    @pl.when(pl.program_id(0) == 0)
    def _():
        xw_s[...] = jnp.dot(x_ref[...], w1_ref[...],
                            preferred_element_type=jnp.float32)

    acc = jnp.dot(f_ref[...], xw_s[...], preferred_element_type=jnp.float32)
    h_ref[...] = jnp.maximum(acc + b1_ref[...], 0.0)


def _final_body(p_ref, w2_ref, b2_ref, o_ref):
    pooled = jnp.max(p_ref[...], axis=0)
    o_ref[...] = jnp.dot(pooled, w2_ref[...],
                         preferred_element_type=jnp.float32) + b2_ref[...]


def _sc_pool_body(h_hbm, ind_hbm, part_hbm, rows_v, ind_v, rows_t, ind_t, pbuf, sem):
    wid = lax.axis_index("s") * 2 + lax.axis_index("c")
    base = wid * CH
    cp_rows = pltpu.async_copy(h_hbm.at[pl.ds(base, CH)], rows_v, sem)
    cp_ind = pltpu.async_copy(ind_hbm.at[pl.ds(base, CH)], ind_v, sem)

    zero = jnp.zeros((L,), jnp.float32)

    @pl.loop(0, B)
    def _(g):
        pbuf[g, pl.ds(0, L)] = zero
        pbuf[g, pl.ds(L, L)] = zero

    cp_rows.wait()
    cp_ind.wait()

    def rmw_row(g, rows, r):
        pbuf[g, pl.ds(0, L)] = jnp.maximum(pbuf[g, pl.ds(0, L)],
                                           rows[r, pl.ds(0, L)])
        pbuf[g, pl.ds(L, L)] = jnp.maximum(pbuf[g, pl.ds(L, L)],
                                           rows[r, pl.ds(L, L)])

    @pl.loop(0, (CH + L - 1) // L)
    def _(k):
        o = jnp.minimum(k * L, CH - L)
        gv = ind_v[pl.ds(o, L)]
        for j in range(L):
            rmw_row(gv[j], rows_v, o + j)

    @pl.when(wid == 0)
    def _():
        pltpu.async_copy(h_hbm.at[pl.ds(TAIL_BASE, TAIL)], rows_t, sem).wait()
        pltpu.async_copy(ind_hbm.at[pl.ds(TAIL_BASE, TAIL)], ind_t, sem).wait()
        gv = ind_t[pl.ds(0, L)]
        for j in range(TAIL):
            rmw_row(gv[j], rows_t, j)

    pltpu.async_copy(pbuf, part_hbm.at[wid], sem).wait()


@functools.cache
def _make_sc_pool():
    return pl.kernel(
        _sc_pool_body,
        out_type=jax.ShapeDtypeStruct((NW, B, HF), jnp.float32),
        mesh=plsc.VectorSubcoreMesh(core_axis_name="c", subcore_axis_name="s"),
        scratch_types=[
            pltpu.VMEM((CH, HF), jnp.float32),
            pltpu.VMEM((CH,), jnp.int32),
            pltpu.VMEM((TAIL, HF), jnp.float32),
            pltpu.VMEM((TAIL,), jnp.int32),
            pltpu.VMEM((B, HF), jnp.float32),
            pltpu.SemaphoreType.DMA,
        ],
    )


def kernel(filtre, X, node_indicator, W1, b1, W2, b2):
    h = pl.pallas_call(
        _main_body,
        grid=(N // BR,),
        in_specs=[
            pl.BlockSpec((N, D), lambda i: (0, 0)),
            pl.BlockSpec((D, HF), lambda i: (0, 0)),
            pl.BlockSpec((BR, N), lambda i: (i, 0)),
            pl.BlockSpec((1, HF), lambda i: (0, 0)),
        ],
        out_specs=pl.BlockSpec((BR, HF), lambda i: (i, 0)),
        out_shape=jax.ShapeDtypeStruct((N, HF), jnp.float32),
        scratch_shapes=[pltpu.VMEM((N, HF), jnp.float32)],
    )(X, W1, filtre, b1.reshape(1, HF))

    part = _make_sc_pool()(h, node_indicator)

    out = pl.pallas_call(
        _final_body,
        out_shape=jax.ShapeDtypeStruct((B, C), jnp.float32),
    )(part, W2, b2.reshape(1, C))
    return out
